# baseline (speedup 1.0000x reference)
import functools

import jax
import jax.numpy as jnp
from jax import lax
from jax.experimental import pallas as pl
from jax.experimental.pallas import tpu as pltpu
from jax.experimental.pallas import tpu_sc as plsc

INPUT_DIM = 768
OUTPUT_DIM = 768
HIDDEN = 128
NUM_EXPERTS = 64
N_TOKENS = 4096

TILE = 128
NT = (N_TOKENS * 2) // TILE + NUM_EXPERTS
P = NT * TILE
CBLK = 256

SC_CORES = 2
SC_SUBCORES = 16
SC_WORKERS = SC_CORES * SC_SUBCORES
TOK_W = N_TOKENS // SC_WORKERS


def _route_body(x_ref, wg_ref,
                aux_ref, pos_ref, ww_ref, te_ref, fetch_ref, oidx_ref,
                a_s, r_s):
    x = x_ref[...]
    logits = jnp.dot(x, wg_ref[...], preferred_element_type=jnp.float32)
    m = jnp.max(logits, axis=1, keepdims=True)
    ex = jnp.exp(logits - m)
    gates = ex / jnp.sum(ex, axis=1, keepdims=True)
    importance = jnp.mean(gates, axis=0)
    tgt = 1.0 / NUM_EXPERTS
    aux_ref[...] = jnp.sum(tgt * (jnp.log(tgt) - jnp.log(importance))).reshape(1, 1)
    eids = lax.broadcasted_iota(jnp.int32, logits.shape, 1)
    is0 = logits == m
    idx0 = jnp.min(jnp.where(is0, eids, NUM_EXPERTS), axis=1, keepdims=True)
    logits1 = jnp.where(eids == idx0, -jnp.inf, logits)
    m1 = jnp.max(logits1, axis=1, keepdims=True)
    is1 = logits1 == m1
    idx1 = jnp.min(jnp.where(is1, eids, NUM_EXPERTS), axis=1, keepdims=True)
    w0 = 1.0 / (1.0 + jnp.exp(m1 - m))
    ww_ref[0] = jnp.broadcast_to(w0, (N_TOKENS, 128))
    ww_ref[1] = jnp.broadcast_to(1.0 - w0, (N_TOKENS, 128))
    a_s[...] = jnp.where((eids == idx0) | (eids == idx1), 1.0, 0.0)
    ri = lax.broadcasted_iota(jnp.int32, (CBLK, CBLK), 0)
    ci = lax.broadcasted_iota(jnp.int32, (CBLK, CBLK), 1)
    ltri = jnp.where(ci < ri, 1.0, 0.0)

    def blk(j, base):
        off = pl.multiple_of(j * CBLK, CBLK)
        ab = a_s[pl.ds(off, CBLK), :]
        r_s[pl.ds(off, CBLK), :] = (
            jnp.dot(ltri, ab, preferred_element_type=jnp.float32) + base)
        return base + jnp.sum(ab, axis=0, keepdims=True)

    counts = lax.fori_loop(0, N_TOKENS // CBLK, blk,
                           jnp.zeros((1, NUM_EXPERTS), jnp.float32))
    pci = ((counts.astype(jnp.int32) + (TILE - 1)) // TILE) * TILE
    pc = pci.astype(jnp.float32)
    fe = lax.broadcasted_iota(jnp.int32, (NUM_EXPERTS, NUM_EXPERTS), 0)
    ee = lax.broadcasted_iota(jnp.int32, (NUM_EXPERTS, NUM_EXPERTS), 1)
    excl = jnp.where(fe < ee, 1.0, 0.0)
    astart = jnp.dot(pc, excl, preferred_element_type=jnp.float32)
    r = r_s[...]
    asb = jnp.broadcast_to(astart, (N_TOKENS, NUM_EXPERTS))
    pos0 = jnp.sum(jnp.where(eids == idx0, r + asb, 0.0), axis=1, keepdims=True)
    pos1 = jnp.sum(jnp.where(eids == idx1, r + asb, 0.0), axis=1, keepdims=True)
    pos_ref[...] = jnp.concatenate([pos0, pos1], axis=1).astype(jnp.int32)
    ident = jnp.where(fe == ee, 1.0, 0.0)
    astart_col = lax.dot_general(ident, astart, (((1,), (1,)), ((), ())),
                                 preferred_element_type=jnp.float32)
    t_iota = lax.broadcasted_iota(jnp.int32, (1, NT), 1)
    tstart = (t_iota * TILE).astype(jnp.float32)
    te = jnp.sum(jnp.where(astart_col <= tstart, 1.0, 0.0),
                 axis=0, keepdims=True) - 1.0
    te_ref[...] = te.astype(jnp.int32)
    total = jnp.sum(pc)
    used = jnp.where(tstart < total, 1, 0)
    fetch_ref[...] = t_iota * used
    oidx_ref[...] = t_iota * used + NT * (1 - used)


def _route(x, w_gate):
    return pl.pallas_call(
        _route_body,
        out_shape=[
            jax.ShapeDtypeStruct((1, 1), jnp.float32),
            jax.ShapeDtypeStruct((N_TOKENS, 2), jnp.int32),
            jax.ShapeDtypeStruct((2, N_TOKENS, 128), jnp.float32),
            jax.ShapeDtypeStruct((1, NT), jnp.int32),
            jax.ShapeDtypeStruct((1, NT), jnp.int32),
            jax.ShapeDtypeStruct((1, NT), jnp.int32),
        ],
        scratch_shapes=[
            pltpu.VMEM((N_TOKENS, NUM_EXPERTS), jnp.float32),
            pltpu.VMEM((N_TOKENS, NUM_EXPERTS), jnp.float32),
        ],
    )(x, w_gate)


def _sc_scatter(x, pos_sc, ww):
    mesh = plsc.VectorSubcoreMesh(core_axis_name="c", subcore_axis_name="s")

    @functools.partial(
        pl.kernel, mesh=mesh,
        out_type=[
            jax.ShapeDtypeStruct((P, INPUT_DIM), jnp.float32),
            jax.ShapeDtypeStruct((P, 128), jnp.float32),
        ],
        scratch_types=[
            pltpu.VMEM((TOK_W,), jnp.int32),
            pltpu.VMEM((TOK_W,), jnp.int32),
            pltpu.VMEM((TOK_W, INPUT_DIM), jnp.float32),
            pltpu.VMEM((TOK_W, 128), jnp.float32),
            pltpu.SemaphoreType.DMA,
        ],
    )
    def k(x_hbm, pos_hbm, ww_hbm, xs_hbm, ws_hbm,
          idx0_v, idx1_v, rows_v, wrow_v, sem):
        wid = lax.axis_index("s") * SC_CORES + lax.axis_index("c")
        base = wid * TOK_W
        pltpu.sync_copy(x_hbm.at[pl.ds(base, TOK_W)], rows_v)
        pltpu.sync_copy(pos_hbm.at[0, wid], idx0_v)
        pltpu.sync_copy(pos_hbm.at[1, wid], idx1_v)
        pltpu.async_copy(rows_v, xs_hbm.at[idx0_v], sem).wait()
        pltpu.async_copy(rows_v, xs_hbm.at[idx1_v], sem).wait()
        pltpu.sync_copy(ww_hbm.at[0, pl.ds(base, TOK_W)], wrow_v)
        pltpu.async_copy(wrow_v, ws_hbm.at[idx0_v], sem).wait()
        pltpu.sync_copy(ww_hbm.at[1, pl.ds(base, TOK_W)], wrow_v)
        pltpu.async_copy(wrow_v, ws_hbm.at[idx1_v], sem).wait()

    return k(x, pos_sc, ww)


def _group_body(te_ref, fetch_ref, oidx_ref,
                xs_ref, ws_ref, W1_ref, b1_ref, W2_ref, b2_ref, ys_ref):
    t = pl.program_id(0)

    @pl.when(oidx_ref[t] == t)
    def _():
        xb = xs_ref[...].astype(jnp.bfloat16)
        h = jnp.dot(xb, W1_ref[0], preferred_element_type=jnp.float32)
        h = jnp.maximum(h + b1_ref[0], 0.0).astype(jnp.bfloat16)
        y = jnp.dot(h, W2_ref[0], preferred_element_type=jnp.float32)
        ys_ref[...] = (y + b2_ref[0]) * ws_ref[:, 0:1]


def _grouped_mlp(te, fetch, oidx, xs, ws, W1, b1r, W2, b2r):
    spec = pltpu.PrefetchScalarGridSpec(
        num_scalar_prefetch=3,
        grid=(NT,),
        in_specs=[
            pl.BlockSpec((TILE, INPUT_DIM), lambda t, te, f, o: (f[t], 0)),
            pl.BlockSpec((TILE, 128), lambda t, te, f, o: (f[t], 0)),
            pl.BlockSpec((1, INPUT_DIM, HIDDEN), lambda t, te, f, o: (te[t], 0, 0)),
            pl.BlockSpec((1, 1, HIDDEN), lambda t, te, f, o: (te[t], 0, 0)),
            pl.BlockSpec((1, HIDDEN, OUTPUT_DIM), lambda t, te, f, o: (te[t], 0, 0)),
            pl.BlockSpec((1, 1, OUTPUT_DIM), lambda t, te, f, o: (te[t], 0, 0)),
        ],
        out_specs=pl.BlockSpec((TILE, OUTPUT_DIM), lambda t, te, f, o: (o[t], 0)),
    )
    return pl.pallas_call(
        _group_body,
        grid_spec=spec,
        out_shape=jax.ShapeDtypeStruct((P + TILE, OUTPUT_DIM), jnp.float32),
    )(te, fetch, oidx, xs, ws, W1, b1r, W2, b2r)


def _sc_gather(ys, pos_sc):
    mesh = plsc.VectorSubcoreMesh(core_axis_name="c", subcore_axis_name="s")
    CH = TOK_W // 2

    @functools.partial(
        pl.kernel, mesh=mesh,
        out_type=jax.ShapeDtypeStruct((N_TOKENS, OUTPUT_DIM), jnp.float32),
        scratch_types=[
            pltpu.VMEM((CH,), jnp.int32),
            pltpu.VMEM((CH,), jnp.int32),
            pltpu.VMEM((CH, OUTPUT_DIM), jnp.float32),
            pltpu.VMEM((CH, OUTPUT_DIM), jnp.float32),
            pltpu.SemaphoreType.DMA,
        ],
    )
    def k(ys_hbm, pos_hbm, out_hbm, idx0_v, idx1_v, r0_v, r1_v, sem):
        wid = lax.axis_index("s") * SC_CORES + lax.axis_index("c")
        for c in range(2):
            base = wid * TOK_W + c * CH
            pltpu.sync_copy(pos_hbm.at[0, wid, pl.ds(c * CH, CH)], idx0_v)
            pltpu.sync_copy(pos_hbm.at[1, wid, pl.ds(c * CH, CH)], idx1_v)
            pltpu.async_copy(ys_hbm.at[idx0_v], r0_v, sem).wait()
            pltpu.async_copy(ys_hbm.at[idx1_v], r1_v, sem).wait()

            def row(i, carry):
                for j in range(OUTPUT_DIM // 16):
                    sl = (i, pl.ds(j * 16, 16))
                    r0_v[sl] = r0_v[sl] + r1_v[sl]
                return carry

            lax.fori_loop(0, CH, row, 0)
            pltpu.sync_copy(r0_v, out_hbm.at[pl.ds(base, CH)])

    return k(ys, pos_sc)


@jax.jit
def kernel(x, W1, b1, W2, b2, w_gate):
    aux, pos, ww, te, fetch, oidx = _route(x, w_gate)
    pos_sc = pos.T.reshape(2, SC_WORKERS, TOK_W)
    xs, ws = _sc_scatter(x, pos_sc, ww)
    ys = _grouped_mlp(te.reshape(NT), fetch.reshape(NT), oidx.reshape(NT),
                      xs, ws,
                      W1.astype(jnp.bfloat16), b1.reshape(NUM_EXPERTS, 1, HIDDEN),
                      W2.astype(jnp.bfloat16), b2.reshape(NUM_EXPERTS, 1, OUTPUT_DIM))
    out = _sc_gather(ys, pos_sc)
    return out, aux[0, 0]


# speedup vs baseline: 1.0762x; 1.0762x over previous
import functools

import jax
import jax.numpy as jnp
from jax import lax
from jax.experimental import pallas as pl
from jax.experimental.pallas import tpu as pltpu
from jax.experimental.pallas import tpu_sc as plsc

INPUT_DIM = 768
OUTPUT_DIM = 768
HIDDEN = 128
NUM_EXPERTS = 64
N_TOKENS = 4096

TILE = 128
NT = (N_TOKENS * 2) // TILE + NUM_EXPERTS
P = NT * TILE
CBLK = 256

SC_CORES = 2
SC_SUBCORES = 16
SC_WORKERS = SC_CORES * SC_SUBCORES
TOK_W = N_TOKENS // SC_WORKERS


def _route_body(x_ref, wg_ref,
                aux_ref, pos_ref, ww_ref, te_ref, fetch_ref, oidx_ref,
                a_s, r_s):
    x = x_ref[...]
    logits = jnp.dot(x, wg_ref[...], preferred_element_type=jnp.float32)
    m = jnp.max(logits, axis=1, keepdims=True)
    ex = jnp.exp(logits - m)
    gates = ex / jnp.sum(ex, axis=1, keepdims=True)
    importance = jnp.mean(gates, axis=0)
    tgt = 1.0 / NUM_EXPERTS
    aux_ref[...] = jnp.sum(tgt * (jnp.log(tgt) - jnp.log(importance))).reshape(1, 1)
    eids = lax.broadcasted_iota(jnp.int32, logits.shape, 1)
    is0 = logits == m
    idx0 = jnp.min(jnp.where(is0, eids, NUM_EXPERTS), axis=1, keepdims=True)
    logits1 = jnp.where(eids == idx0, -jnp.inf, logits)
    m1 = jnp.max(logits1, axis=1, keepdims=True)
    is1 = logits1 == m1
    idx1 = jnp.min(jnp.where(is1, eids, NUM_EXPERTS), axis=1, keepdims=True)
    w0 = 1.0 / (1.0 + jnp.exp(m1 - m))
    ww_ref[0] = jnp.broadcast_to(w0, (N_TOKENS, 128))
    ww_ref[1] = jnp.broadcast_to(1.0 - w0, (N_TOKENS, 128))
    a_s[...] = jnp.where((eids == idx0) | (eids == idx1), 1.0, 0.0)
    ri = lax.broadcasted_iota(jnp.int32, (CBLK, CBLK), 0)
    ci = lax.broadcasted_iota(jnp.int32, (CBLK, CBLK), 1)
    ltri = jnp.where(ci < ri, 1.0, 0.0)

    def blk(j, base):
        off = pl.multiple_of(j * CBLK, CBLK)
        ab = a_s[pl.ds(off, CBLK), :]
        r_s[pl.ds(off, CBLK), :] = (
            jnp.dot(ltri, ab, preferred_element_type=jnp.float32) + base)
        return base + jnp.sum(ab, axis=0, keepdims=True)

    counts = lax.fori_loop(0, N_TOKENS // CBLK, blk,
                           jnp.zeros((1, NUM_EXPERTS), jnp.float32))
    pci = ((counts.astype(jnp.int32) + (TILE - 1)) // TILE) * TILE
    pc = pci.astype(jnp.float32)
    fe = lax.broadcasted_iota(jnp.int32, (NUM_EXPERTS, NUM_EXPERTS), 0)
    ee = lax.broadcasted_iota(jnp.int32, (NUM_EXPERTS, NUM_EXPERTS), 1)
    excl = jnp.where(fe < ee, 1.0, 0.0)
    astart = jnp.dot(pc, excl, preferred_element_type=jnp.float32)
    r = r_s[...]
    asb = jnp.broadcast_to(astart, (N_TOKENS, NUM_EXPERTS))
    pos0 = jnp.sum(jnp.where(eids == idx0, r + asb, 0.0), axis=1, keepdims=True)
    pos1 = jnp.sum(jnp.where(eids == idx1, r + asb, 0.0), axis=1, keepdims=True)
    pos_ref[...] = jnp.concatenate([pos0, pos1], axis=1).astype(jnp.int32)
    ident = jnp.where(fe == ee, 1.0, 0.0)
    astart_col = lax.dot_general(ident, astart, (((1,), (1,)), ((), ())),
                                 preferred_element_type=jnp.float32)
    t_iota = lax.broadcasted_iota(jnp.int32, (1, NT), 1)
    tstart = (t_iota * TILE).astype(jnp.float32)
    te = jnp.sum(jnp.where(astart_col <= tstart, 1.0, 0.0),
                 axis=0, keepdims=True) - 1.0
    te_ref[...] = te.astype(jnp.int32)
    total = jnp.sum(pc)
    used = jnp.where(tstart < total, 1, 0)
    fetch_ref[...] = t_iota * used
    oidx_ref[...] = t_iota * used + NT * (1 - used)


def _route(x, w_gate):
    return pl.pallas_call(
        _route_body,
        out_shape=[
            jax.ShapeDtypeStruct((1, 1), jnp.float32),
            jax.ShapeDtypeStruct((N_TOKENS, 2), jnp.int32),
            jax.ShapeDtypeStruct((2, N_TOKENS, 128), jnp.float32),
            jax.ShapeDtypeStruct((1, NT), jnp.int32),
            jax.ShapeDtypeStruct((1, NT), jnp.int32),
            jax.ShapeDtypeStruct((1, NT), jnp.int32),
        ],
        scratch_shapes=[
            pltpu.VMEM((N_TOKENS, NUM_EXPERTS), jnp.float32),
            pltpu.VMEM((N_TOKENS, NUM_EXPERTS), jnp.float32),
        ],
    )(x, w_gate)


def _sc_scatter(x, pos_sc, ww):
    mesh = plsc.VectorSubcoreMesh(core_axis_name="c", subcore_axis_name="s")

    @functools.partial(
        pl.kernel, mesh=mesh,
        out_type=[
            jax.ShapeDtypeStruct((P, INPUT_DIM), jnp.float32),
            jax.ShapeDtypeStruct((P, 128), jnp.float32),
        ],
        scratch_types=[
            pltpu.VMEM((TOK_W,), jnp.int32),
            pltpu.VMEM((TOK_W,), jnp.int32),
            pltpu.VMEM((TOK_W, INPUT_DIM), jnp.float32),
            pltpu.VMEM((TOK_W, 128), jnp.float32),
            pltpu.SemaphoreType.DMA,
        ],
    )
    def k(x_hbm, pos_hbm, ww_hbm, xs_hbm, ws_hbm,
          idx0_v, idx1_v, rows_v, wrow_v, sem):
        wid = lax.axis_index("s") * SC_CORES + lax.axis_index("c")
        base = wid * TOK_W
        pltpu.sync_copy(x_hbm.at[pl.ds(base, TOK_W)], rows_v)
        pltpu.sync_copy(pos_hbm.at[0, wid], idx0_v)
        pltpu.sync_copy(pos_hbm.at[1, wid], idx1_v)
        pltpu.async_copy(rows_v, xs_hbm.at[idx0_v], sem).wait()
        pltpu.async_copy(rows_v, xs_hbm.at[idx1_v], sem).wait()
        pltpu.sync_copy(ww_hbm.at[0, pl.ds(base, TOK_W)], wrow_v)
        pltpu.async_copy(wrow_v, ws_hbm.at[idx0_v], sem).wait()
        pltpu.sync_copy(ww_hbm.at[1, pl.ds(base, TOK_W)], wrow_v)
        pltpu.async_copy(wrow_v, ws_hbm.at[idx1_v], sem).wait()

    return k(x, pos_sc, ww)


def _group_body(te_ref, fetch_ref, oidx_ref,
                xs_ref, ws_ref, W1_ref, b1_ref, W2_ref, b2_ref, ys_ref):
    t = pl.program_id(0)

    @pl.when(oidx_ref[t] == t)
    def _():
        h = jnp.dot(xs_ref[...], W1_ref[0], preferred_element_type=jnp.float32)
        h = jnp.maximum(h + b1_ref[0], 0.0)
        y = jnp.dot(h, W2_ref[0], preferred_element_type=jnp.float32)
        ys_ref[...] = (y + b2_ref[0]) * ws_ref[:, 0:1]


def _grouped_mlp(te, fetch, oidx, xs, ws, W1, b1r, W2, b2r):
    spec = pltpu.PrefetchScalarGridSpec(
        num_scalar_prefetch=3,
        grid=(NT,),
        in_specs=[
            pl.BlockSpec((TILE, INPUT_DIM), lambda t, te, f, o: (f[t], 0)),
            pl.BlockSpec((TILE, 128), lambda t, te, f, o: (f[t], 0)),
            pl.BlockSpec((1, INPUT_DIM, HIDDEN), lambda t, te, f, o: (te[t], 0, 0)),
            pl.BlockSpec((1, 1, HIDDEN), lambda t, te, f, o: (te[t], 0, 0)),
            pl.BlockSpec((1, HIDDEN, OUTPUT_DIM), lambda t, te, f, o: (te[t], 0, 0)),
            pl.BlockSpec((1, 1, OUTPUT_DIM), lambda t, te, f, o: (te[t], 0, 0)),
        ],
        out_specs=pl.BlockSpec((TILE, OUTPUT_DIM), lambda t, te, f, o: (o[t], 0)),
    )
    return pl.pallas_call(
        _group_body,
        grid_spec=spec,
        out_shape=jax.ShapeDtypeStruct((P + TILE, OUTPUT_DIM), jnp.float32),
    )(te, fetch, oidx, xs, ws, W1, b1r, W2, b2r)


def _sc_gather(ys, pos_sc):
    mesh = plsc.VectorSubcoreMesh(core_axis_name="c", subcore_axis_name="s")
    CH = TOK_W // 2

    @functools.partial(
        pl.kernel, mesh=mesh,
        out_type=jax.ShapeDtypeStruct((N_TOKENS, OUTPUT_DIM), jnp.float32),
        scratch_types=[
            pltpu.VMEM((CH,), jnp.int32),
            pltpu.VMEM((CH,), jnp.int32),
            pltpu.VMEM((CH, OUTPUT_DIM), jnp.float32),
            pltpu.VMEM((CH, OUTPUT_DIM), jnp.float32),
            pltpu.SemaphoreType.DMA,
        ],
    )
    def k(ys_hbm, pos_hbm, out_hbm, idx0_v, idx1_v, r0_v, r1_v, sem):
        wid = lax.axis_index("s") * SC_CORES + lax.axis_index("c")
        for c in range(2):
            base = wid * TOK_W + c * CH
            pltpu.sync_copy(pos_hbm.at[0, wid, pl.ds(c * CH, CH)], idx0_v)
            pltpu.sync_copy(pos_hbm.at[1, wid, pl.ds(c * CH, CH)], idx1_v)
            pltpu.async_copy(ys_hbm.at[idx0_v], r0_v, sem).wait()
            pltpu.async_copy(ys_hbm.at[idx1_v], r1_v, sem).wait()

            def row(i, carry):
                for j in range(OUTPUT_DIM // 16):
                    sl = (i, pl.ds(j * 16, 16))
                    r0_v[sl] = r0_v[sl] + r1_v[sl]
                return carry

            lax.fori_loop(0, CH, row, 0)
            pltpu.sync_copy(r0_v, out_hbm.at[pl.ds(base, CH)])

    return k(ys, pos_sc)


@jax.jit
def kernel(x, W1, b1, W2, b2, w_gate):
    aux, pos, ww, te, fetch, oidx = _route(x, w_gate)
    pos_sc = pos.T.reshape(2, SC_WORKERS, TOK_W)
    xs, ws = _sc_scatter(x, pos_sc, ww)
    ys = _grouped_mlp(te.reshape(NT), fetch.reshape(NT), oidx.reshape(NT),
                      xs, ws,
                      W1, b1.reshape(NUM_EXPERTS, 1, HIDDEN),
                      W2, b2.reshape(NUM_EXPERTS, 1, OUTPUT_DIM))
    out = _sc_gather(ys, pos_sc)
    return out, aux[0, 0]


# speedup vs baseline: 1.1111x; 1.0325x over previous
import functools

import jax
import jax.numpy as jnp
from jax import lax
from jax.experimental import pallas as pl
from jax.experimental.pallas import tpu as pltpu
from jax.experimental.pallas import tpu_sc as plsc

INPUT_DIM = 768
OUTPUT_DIM = 768
HIDDEN = 128
NUM_EXPERTS = 64
N_TOKENS = 4096

TILE = 128
NT = (N_TOKENS * 2) // TILE + NUM_EXPERTS
P = NT * TILE
CBLK = 256

SC_CORES = 2
SC_SUBCORES = 16
SC_WORKERS = SC_CORES * SC_SUBCORES
TOK_W = N_TOKENS // SC_WORKERS


def _route_body(x_ref, wg_ref,
                aux_ref, pos_ref, ww_ref, te_ref, fetch_ref, oidx_ref,
                a_s, r_s):
    x = x_ref[...]
    logits = jnp.dot(x, wg_ref[...], preferred_element_type=jnp.float32)
    m = jnp.max(logits, axis=1, keepdims=True)
    ex = jnp.exp(logits - m)
    gates = ex / jnp.sum(ex, axis=1, keepdims=True)
    importance = jnp.mean(gates, axis=0)
    tgt = 1.0 / NUM_EXPERTS
    aux_ref[...] = jnp.sum(tgt * (jnp.log(tgt) - jnp.log(importance))).reshape(1, 1)
    eids = lax.broadcasted_iota(jnp.int32, logits.shape, 1)
    is0 = logits == m
    idx0 = jnp.min(jnp.where(is0, eids, NUM_EXPERTS), axis=1, keepdims=True)
    logits1 = jnp.where(eids == idx0, -jnp.inf, logits)
    m1 = jnp.max(logits1, axis=1, keepdims=True)
    is1 = logits1 == m1
    idx1 = jnp.min(jnp.where(is1, eids, NUM_EXPERTS), axis=1, keepdims=True)
    w0 = 1.0 / (1.0 + jnp.exp(m1 - m))
    ww_ref[0] = jnp.broadcast_to(w0, (N_TOKENS, 128))
    ww_ref[1] = jnp.broadcast_to(1.0 - w0, (N_TOKENS, 128))
    a_s[...] = jnp.where((eids == idx0) | (eids == idx1), 1.0, 0.0)
    ri = lax.broadcasted_iota(jnp.int32, (CBLK, CBLK), 0)
    ci = lax.broadcasted_iota(jnp.int32, (CBLK, CBLK), 1)
    ltri = jnp.where(ci < ri, 1.0, 0.0)

    def blk(j, base):
        off = pl.multiple_of(j * CBLK, CBLK)
        ab = a_s[pl.ds(off, CBLK), :]
        r_s[pl.ds(off, CBLK), :] = (
            jnp.dot(ltri, ab, preferred_element_type=jnp.float32) + base)
        return base + jnp.sum(ab, axis=0, keepdims=True)

    counts = lax.fori_loop(0, N_TOKENS // CBLK, blk,
                           jnp.zeros((1, NUM_EXPERTS), jnp.float32))
    pci = ((counts.astype(jnp.int32) + (TILE - 1)) // TILE) * TILE
    pc = pci.astype(jnp.float32)
    fe = lax.broadcasted_iota(jnp.int32, (NUM_EXPERTS, NUM_EXPERTS), 0)
    ee = lax.broadcasted_iota(jnp.int32, (NUM_EXPERTS, NUM_EXPERTS), 1)
    excl = jnp.where(fe < ee, 1.0, 0.0)
    astart = jnp.dot(pc, excl, preferred_element_type=jnp.float32)
    r = r_s[...]
    asb = jnp.broadcast_to(astart, (N_TOKENS, NUM_EXPERTS))
    pos0 = jnp.sum(jnp.where(eids == idx0, r + asb, 0.0), axis=1, keepdims=True)
    pos1 = jnp.sum(jnp.where(eids == idx1, r + asb, 0.0), axis=1, keepdims=True)
    pos_ref[...] = jnp.concatenate([pos0, pos1], axis=1).astype(jnp.int32)
    ident = jnp.where(fe == ee, 1.0, 0.0)
    astart_col = lax.dot_general(ident, astart, (((1,), (1,)), ((), ())),
                                 preferred_element_type=jnp.float32)
    t_iota = lax.broadcasted_iota(jnp.int32, (1, NT), 1)
    tstart = (t_iota * TILE).astype(jnp.float32)
    te = jnp.sum(jnp.where(astart_col <= tstart, 1.0, 0.0),
                 axis=0, keepdims=True) - 1.0
    te_ref[...] = te.astype(jnp.int32)
    total = jnp.sum(pc)
    used = jnp.where(tstart < total, 1, 0)
    fetch_ref[...] = t_iota * used
    oidx_ref[...] = t_iota * used + NT * (1 - used)


def _route(x, w_gate):
    return pl.pallas_call(
        _route_body,
        out_shape=[
            jax.ShapeDtypeStruct((1, 1), jnp.float32),
            jax.ShapeDtypeStruct((N_TOKENS, 2), jnp.int32),
            jax.ShapeDtypeStruct((2, N_TOKENS, 128), jnp.float32),
            jax.ShapeDtypeStruct((1, NT), jnp.int32),
            jax.ShapeDtypeStruct((1, NT), jnp.int32),
            jax.ShapeDtypeStruct((1, NT), jnp.int32),
        ],
        scratch_shapes=[
            pltpu.VMEM((N_TOKENS, NUM_EXPERTS), jnp.float32),
            pltpu.VMEM((N_TOKENS, NUM_EXPERTS), jnp.float32),
        ],
    )(x, w_gate)


def _sc_scatter(x, pos_sc, ww):
    mesh = plsc.VectorSubcoreMesh(core_axis_name="c", subcore_axis_name="s")

    @functools.partial(
        pl.kernel, mesh=mesh,
        out_type=[
            jax.ShapeDtypeStruct((P, INPUT_DIM), jnp.float32),
            jax.ShapeDtypeStruct((P, 128), jnp.float32),
        ],
        scratch_types=[
            pltpu.VMEM((TOK_W,), jnp.int32),
            pltpu.VMEM((TOK_W,), jnp.int32),
            pltpu.VMEM((TOK_W, INPUT_DIM), jnp.float32),
            pltpu.VMEM((TOK_W, 128), jnp.float32),
            pltpu.SemaphoreType.DMA,
            pltpu.SemaphoreType.DMA,
            pltpu.SemaphoreType.DMA,
        ],
    )
    def k(x_hbm, pos_hbm, ww_hbm, xs_hbm, ws_hbm,
          idx0_v, idx1_v, rows_v, wrow_v, sem_i, sem_s, sem_w):
        wid = lax.axis_index("s") * SC_CORES + lax.axis_index("c")
        base = wid * TOK_W
        hx = pltpu.async_copy(x_hbm.at[pl.ds(base, TOK_W)], rows_v, sem_i)
        pltpu.sync_copy(pos_hbm.at[0, wid], idx0_v)
        pltpu.sync_copy(pos_hbm.at[1, wid], idx1_v)
        hx.wait()
        s1 = pltpu.async_copy(rows_v, xs_hbm.at[idx0_v], sem_s)
        s2 = pltpu.async_copy(rows_v, xs_hbm.at[idx1_v], sem_s)
        pltpu.sync_copy(ww_hbm.at[0, pl.ds(base, TOK_W)], wrow_v)
        s3 = pltpu.async_copy(wrow_v, ws_hbm.at[idx0_v], sem_w)
        s3.wait()
        pltpu.sync_copy(ww_hbm.at[1, pl.ds(base, TOK_W)], wrow_v)
        s4 = pltpu.async_copy(wrow_v, ws_hbm.at[idx1_v], sem_w)
        s1.wait()
        s2.wait()
        s4.wait()

    return k(x, pos_sc, ww)


def _group_body(te_ref, fetch_ref, oidx_ref,
                xs_ref, ws_ref, W1_ref, b1_ref, W2_ref, b2_ref, ys_ref):
    t = pl.program_id(0)

    @pl.when(oidx_ref[t] == t)
    def _():
        h = jnp.dot(xs_ref[...], W1_ref[0], preferred_element_type=jnp.float32)
        h = jnp.maximum(h + b1_ref[0], 0.0)
        y = jnp.dot(h, W2_ref[0], preferred_element_type=jnp.float32)
        ys_ref[...] = (y + b2_ref[0]) * ws_ref[:, 0:1]


def _grouped_mlp(te, fetch, oidx, xs, ws, W1, b1r, W2, b2r):
    spec = pltpu.PrefetchScalarGridSpec(
        num_scalar_prefetch=3,
        grid=(NT,),
        in_specs=[
            pl.BlockSpec((TILE, INPUT_DIM), lambda t, te, f, o: (f[t], 0)),
            pl.BlockSpec((TILE, 128), lambda t, te, f, o: (f[t], 0)),
            pl.BlockSpec((1, INPUT_DIM, HIDDEN), lambda t, te, f, o: (te[t], 0, 0)),
            pl.BlockSpec((1, 1, HIDDEN), lambda t, te, f, o: (te[t], 0, 0)),
            pl.BlockSpec((1, HIDDEN, OUTPUT_DIM), lambda t, te, f, o: (te[t], 0, 0)),
            pl.BlockSpec((1, 1, OUTPUT_DIM), lambda t, te, f, o: (te[t], 0, 0)),
        ],
        out_specs=pl.BlockSpec((TILE, OUTPUT_DIM), lambda t, te, f, o: (o[t], 0)),
    )
    return pl.pallas_call(
        _group_body,
        grid_spec=spec,
        out_shape=jax.ShapeDtypeStruct((P + TILE, OUTPUT_DIM), jnp.float32),
    )(te, fetch, oidx, xs, ws, W1, b1r, W2, b2r)


def _sc_gather(ys, pos_sc):
    mesh = plsc.VectorSubcoreMesh(core_axis_name="c", subcore_axis_name="s")
    CH = 32
    NCH = TOK_W // CH

    @functools.partial(
        pl.kernel, mesh=mesh,
        out_type=jax.ShapeDtypeStruct((N_TOKENS, OUTPUT_DIM), jnp.float32),
        scratch_types=[
            pltpu.VMEM((TOK_W,), jnp.int32),
            pltpu.VMEM((TOK_W,), jnp.int32),
            pltpu.VMEM((CH, OUTPUT_DIM), jnp.float32),
            pltpu.VMEM((CH, OUTPUT_DIM), jnp.float32),
            pltpu.VMEM((CH, OUTPUT_DIM), jnp.float32),
            pltpu.VMEM((CH, OUTPUT_DIM), jnp.float32),
            pltpu.SemaphoreType.DMA,
            pltpu.SemaphoreType.DMA,
            pltpu.SemaphoreType.DMA,
        ],
    )
    def k(ys_hbm, pos_hbm, out_hbm, idx0_v, idx1_v,
          a0, a1, b0, b1, sem_a, sem_b, sem_o):
        wid = lax.axis_index("s") * SC_CORES + lax.axis_index("c")
        base = wid * TOK_W
        pltpu.sync_copy(pos_hbm.at[0, wid], idx0_v)
        pltpu.sync_copy(pos_hbm.at[1, wid], idx1_v)
        bufs = [(a0, a1, sem_a), (b0, b1, sem_b)]

        def issue(c):
            r0, r1, sem = bufs[c % 2]
            h0 = pltpu.async_copy(
                ys_hbm.at[idx0_v.at[pl.ds(c * CH, CH)]], r0, sem)
            h1 = pltpu.async_copy(
                ys_hbm.at[idx1_v.at[pl.ds(c * CH, CH)]], r1, sem)
            return h0, h1

        hs = {0: issue(0)}
        outh = {}
        for c in range(NCH):
            r0, r1, _ = bufs[c % 2]
            if c + 1 < NCH:
                if c >= 1:
                    outh[c - 1].wait()
                hs[c + 1] = issue(c + 1)
            h0, h1 = hs[c]
            h0.wait()
            h1.wait()

            def row(i, carry):
                for j in range(OUTPUT_DIM // 16):
                    sl = (i, pl.ds(j * 16, 16))
                    r0[sl] = r0[sl] + r1[sl]
                return carry

            lax.fori_loop(0, CH, row, 0)
            outh[c] = pltpu.async_copy(
                r0, out_hbm.at[pl.ds(base + c * CH, CH)], sem_o)
        outh[NCH - 2].wait()
        outh[NCH - 1].wait()

    return k(ys, pos_sc)


@jax.jit
def kernel(x, W1, b1, W2, b2, w_gate):
    aux, pos, ww, te, fetch, oidx = _route(x, w_gate)
    pos_sc = pos.T.reshape(2, SC_WORKERS, TOK_W)
    xs, ws = _sc_scatter(x, pos_sc, ww)
    ys = _grouped_mlp(te.reshape(NT), fetch.reshape(NT), oidx.reshape(NT),
                      xs, ws,
                      W1, b1.reshape(NUM_EXPERTS, 1, HIDDEN),
                      W2, b2.reshape(NUM_EXPERTS, 1, OUTPUT_DIM))
    out = _sc_gather(ys, pos_sc)
    return out, aux[0, 0]


# speedup vs baseline: 1.1974x; 1.0776x over previous
import functools

import jax
import jax.numpy as jnp
from jax import lax
from jax.experimental import pallas as pl
from jax.experimental.pallas import tpu as pltpu
from jax.experimental.pallas import tpu_sc as plsc

INPUT_DIM = 768
OUTPUT_DIM = 768
HIDDEN = 128
NUM_EXPERTS = 64
N_TOKENS = 4096

TILE = 256
NT = (N_TOKENS * 2) // TILE + NUM_EXPERTS
P = NT * TILE
CBLK = 256

SC_CORES = 2
SC_SUBCORES = 16
SC_WORKERS = SC_CORES * SC_SUBCORES
TOK_W = N_TOKENS // SC_WORKERS


def _route_body(x_ref, wg_ref,
                aux_ref, pos_ref, ww_ref, te_ref, fetch_ref, oidx_ref,
                a_s, r_s):
    x = x_ref[...]
    logits = jnp.dot(x, wg_ref[...], preferred_element_type=jnp.float32)
    m = jnp.max(logits, axis=1, keepdims=True)
    ex = jnp.exp(logits - m)
    gates = ex / jnp.sum(ex, axis=1, keepdims=True)
    importance = jnp.mean(gates, axis=0)
    tgt = 1.0 / NUM_EXPERTS
    aux_ref[...] = jnp.sum(tgt * (jnp.log(tgt) - jnp.log(importance))).reshape(1, 1)
    eids = lax.broadcasted_iota(jnp.int32, logits.shape, 1)
    is0 = logits == m
    idx0 = jnp.min(jnp.where(is0, eids, NUM_EXPERTS), axis=1, keepdims=True)
    logits1 = jnp.where(eids == idx0, -jnp.inf, logits)
    m1 = jnp.max(logits1, axis=1, keepdims=True)
    is1 = logits1 == m1
    idx1 = jnp.min(jnp.where(is1, eids, NUM_EXPERTS), axis=1, keepdims=True)
    w0 = 1.0 / (1.0 + jnp.exp(m1 - m))
    ww_ref[0] = jnp.broadcast_to(w0, (N_TOKENS, 128))
    ww_ref[1] = jnp.broadcast_to(1.0 - w0, (N_TOKENS, 128))
    a_s[...] = jnp.where((eids == idx0) | (eids == idx1), 1.0, 0.0)
    ri = lax.broadcasted_iota(jnp.int32, (CBLK, CBLK), 0)
    ci = lax.broadcasted_iota(jnp.int32, (CBLK, CBLK), 1)
    ltri = jnp.where(ci < ri, 1.0, 0.0)

    def blk(j, base):
        off = pl.multiple_of(j * CBLK, CBLK)
        ab = a_s[pl.ds(off, CBLK), :]
        r_s[pl.ds(off, CBLK), :] = (
            jnp.dot(ltri, ab, preferred_element_type=jnp.float32) + base)
        return base + jnp.sum(ab, axis=0, keepdims=True)

    counts = lax.fori_loop(0, N_TOKENS // CBLK, blk,
                           jnp.zeros((1, NUM_EXPERTS), jnp.float32))
    pci = ((counts.astype(jnp.int32) + (TILE - 1)) // TILE) * TILE
    pc = pci.astype(jnp.float32)
    fe = lax.broadcasted_iota(jnp.int32, (NUM_EXPERTS, NUM_EXPERTS), 0)
    ee = lax.broadcasted_iota(jnp.int32, (NUM_EXPERTS, NUM_EXPERTS), 1)
    excl = jnp.where(fe < ee, 1.0, 0.0)
    astart = jnp.dot(pc, excl, preferred_element_type=jnp.float32)
    r = r_s[...]
    asb = jnp.broadcast_to(astart, (N_TOKENS, NUM_EXPERTS))
    pos0 = jnp.sum(jnp.where(eids == idx0, r + asb, 0.0), axis=1, keepdims=True)
    pos1 = jnp.sum(jnp.where(eids == idx1, r + asb, 0.0), axis=1, keepdims=True)
    pos_ref[...] = jnp.concatenate([pos0, pos1], axis=1).astype(jnp.int32)
    ident = jnp.where(fe == ee, 1.0, 0.0)
    astart_col = lax.dot_general(ident, astart, (((1,), (1,)), ((), ())),
                                 preferred_element_type=jnp.float32)
    t_iota = lax.broadcasted_iota(jnp.int32, (1, NT), 1)
    tstart = (t_iota * TILE).astype(jnp.float32)
    te = jnp.sum(jnp.where(astart_col <= tstart, 1.0, 0.0),
                 axis=0, keepdims=True) - 1.0
    te_ref[...] = te.astype(jnp.int32)
    total = jnp.sum(pc)
    used = jnp.where(tstart < total, 1, 0)
    fetch_ref[...] = t_iota * used
    oidx_ref[...] = t_iota * used + NT * (1 - used)


def _route(x, w_gate):
    return pl.pallas_call(
        _route_body,
        out_shape=[
            jax.ShapeDtypeStruct((1, 1), jnp.float32),
            jax.ShapeDtypeStruct((N_TOKENS, 2), jnp.int32),
            jax.ShapeDtypeStruct((2, N_TOKENS, 128), jnp.float32),
            jax.ShapeDtypeStruct((1, NT), jnp.int32),
            jax.ShapeDtypeStruct((1, NT), jnp.int32),
            jax.ShapeDtypeStruct((1, NT), jnp.int32),
        ],
        scratch_shapes=[
            pltpu.VMEM((N_TOKENS, NUM_EXPERTS), jnp.float32),
            pltpu.VMEM((N_TOKENS, NUM_EXPERTS), jnp.float32),
        ],
    )(x, w_gate)


def _sc_scatter(x, pos_sc, ww):
    mesh = plsc.VectorSubcoreMesh(core_axis_name="c", subcore_axis_name="s")

    @functools.partial(
        pl.kernel, mesh=mesh,
        out_type=[
            jax.ShapeDtypeStruct((P, INPUT_DIM), jnp.float32),
            jax.ShapeDtypeStruct((P, 128), jnp.float32),
        ],
        scratch_types=[
            pltpu.VMEM((TOK_W,), jnp.int32),
            pltpu.VMEM((TOK_W,), jnp.int32),
            pltpu.VMEM((TOK_W, INPUT_DIM), jnp.float32),
            pltpu.VMEM((TOK_W, 128), jnp.float32),
            pltpu.SemaphoreType.DMA,
            pltpu.SemaphoreType.DMA,
            pltpu.SemaphoreType.DMA,
        ],
    )
    def k(x_hbm, pos_hbm, ww_hbm, xs_hbm, ws_hbm,
          idx0_v, idx1_v, rows_v, wrow_v, sem_i, sem_s, sem_w):
        wid = lax.axis_index("s") * SC_CORES + lax.axis_index("c")
        base = wid * TOK_W
        hx = pltpu.async_copy(x_hbm.at[pl.ds(base, TOK_W)], rows_v, sem_i)
        pltpu.sync_copy(pos_hbm.at[0, wid], idx0_v)
        pltpu.sync_copy(pos_hbm.at[1, wid], idx1_v)
        hx.wait()
        s1 = pltpu.async_copy(rows_v, xs_hbm.at[idx0_v], sem_s)
        s2 = pltpu.async_copy(rows_v, xs_hbm.at[idx1_v], sem_s)
        pltpu.sync_copy(ww_hbm.at[0, pl.ds(base, TOK_W)], wrow_v)
        s3 = pltpu.async_copy(wrow_v, ws_hbm.at[idx0_v], sem_w)
        s3.wait()
        pltpu.sync_copy(ww_hbm.at[1, pl.ds(base, TOK_W)], wrow_v)
        s4 = pltpu.async_copy(wrow_v, ws_hbm.at[idx1_v], sem_w)
        s1.wait()
        s2.wait()
        s4.wait()

    return k(x, pos_sc, ww)


def _group_body(te_ref, fetch_ref, oidx_ref,
                xs_ref, ws_ref, W1_ref, b1_ref, W2_ref, b2_ref, ys_ref):
    t = pl.program_id(0)

    @pl.when(oidx_ref[t] == t)
    def _():
        h = jnp.dot(xs_ref[...], W1_ref[0], preferred_element_type=jnp.float32)
        h = jnp.maximum(h + b1_ref[0], 0.0)
        y = jnp.dot(h, W2_ref[0], preferred_element_type=jnp.float32)
        ys_ref[...] = (y + b2_ref[0]) * ws_ref[:, 0:1]


def _grouped_mlp(te, fetch, oidx, xs, ws, W1, b1r, W2, b2r):
    spec = pltpu.PrefetchScalarGridSpec(
        num_scalar_prefetch=3,
        grid=(NT,),
        in_specs=[
            pl.BlockSpec((TILE, INPUT_DIM), lambda t, te, f, o: (f[t], 0)),
            pl.BlockSpec((TILE, 128), lambda t, te, f, o: (f[t], 0)),
            pl.BlockSpec((1, INPUT_DIM, HIDDEN), lambda t, te, f, o: (te[t], 0, 0)),
            pl.BlockSpec((1, 1, HIDDEN), lambda t, te, f, o: (te[t], 0, 0)),
            pl.BlockSpec((1, HIDDEN, OUTPUT_DIM), lambda t, te, f, o: (te[t], 0, 0)),
            pl.BlockSpec((1, 1, OUTPUT_DIM), lambda t, te, f, o: (te[t], 0, 0)),
        ],
        out_specs=pl.BlockSpec((TILE, OUTPUT_DIM), lambda t, te, f, o: (o[t], 0)),
    )
    return pl.pallas_call(
        _group_body,
        grid_spec=spec,
        out_shape=jax.ShapeDtypeStruct((P + TILE, OUTPUT_DIM), jnp.float32),
    )(te, fetch, oidx, xs, ws, W1, b1r, W2, b2r)


def _sc_gather(ys, pos_sc):
    mesh = plsc.VectorSubcoreMesh(core_axis_name="c", subcore_axis_name="s")
    CH = 32
    NCH = TOK_W // CH

    @functools.partial(
        pl.kernel, mesh=mesh,
        out_type=jax.ShapeDtypeStruct((N_TOKENS, OUTPUT_DIM), jnp.float32),
        scratch_types=[
            pltpu.VMEM((TOK_W,), jnp.int32),
            pltpu.VMEM((TOK_W,), jnp.int32),
            pltpu.VMEM((CH, OUTPUT_DIM), jnp.float32),
            pltpu.VMEM((CH, OUTPUT_DIM), jnp.float32),
            pltpu.VMEM((CH, OUTPUT_DIM), jnp.float32),
            pltpu.VMEM((CH, OUTPUT_DIM), jnp.float32),
            pltpu.SemaphoreType.DMA,
            pltpu.SemaphoreType.DMA,
            pltpu.SemaphoreType.DMA,
        ],
    )
    def k(ys_hbm, pos_hbm, out_hbm, idx0_v, idx1_v,
          a0, a1, b0, b1, sem_a, sem_b, sem_o):
        wid = lax.axis_index("s") * SC_CORES + lax.axis_index("c")
        base = wid * TOK_W
        pltpu.sync_copy(pos_hbm.at[0, wid], idx0_v)
        pltpu.sync_copy(pos_hbm.at[1, wid], idx1_v)
        bufs = [(a0, a1, sem_a), (b0, b1, sem_b)]

        def issue(c):
            r0, r1, sem = bufs[c % 2]
            h0 = pltpu.async_copy(
                ys_hbm.at[idx0_v.at[pl.ds(c * CH, CH)]], r0, sem)
            h1 = pltpu.async_copy(
                ys_hbm.at[idx1_v.at[pl.ds(c * CH, CH)]], r1, sem)
            return h0, h1

        hs = {0: issue(0)}
        outh = {}
        for c in range(NCH):
            r0, r1, _ = bufs[c % 2]
            if c + 1 < NCH:
                if c >= 1:
                    outh[c - 1].wait()
                hs[c + 1] = issue(c + 1)
            h0, h1 = hs[c]
            h0.wait()
            h1.wait()

            def row(i, carry):
                for j in range(OUTPUT_DIM // 16):
                    sl = (i, pl.ds(j * 16, 16))
                    r0[sl] = r0[sl] + r1[sl]
                return carry

            lax.fori_loop(0, CH, row, 0)
            outh[c] = pltpu.async_copy(
                r0, out_hbm.at[pl.ds(base + c * CH, CH)], sem_o)
        outh[NCH - 2].wait()
        outh[NCH - 1].wait()

    return k(ys, pos_sc)


@jax.jit
def kernel(x, W1, b1, W2, b2, w_gate):
    aux, pos, ww, te, fetch, oidx = _route(x, w_gate)
    pos_sc = pos.T.reshape(2, SC_WORKERS, TOK_W)
    xs, ws = _sc_scatter(x, pos_sc, ww)
    ys = _grouped_mlp(te.reshape(NT), fetch.reshape(NT), oidx.reshape(NT),
                      xs, ws,
                      W1, b1.reshape(NUM_EXPERTS, 1, HIDDEN),
                      W2, b2.reshape(NUM_EXPERTS, 1, OUTPUT_DIM))
    out = _sc_gather(ys, pos_sc)
    return out, aux[0, 0]
